# Initial kernel scaffold; baseline (speedup 1.0000x reference)
import jax
import jax.numpy as jnp
from jax.experimental import pallas as pl


def kernel(x, edge_index, W1, b1, W2, b2):
    raise NotImplementedError("write your pallas kernel here")


# speedup vs baseline: 13.3090x; 13.3090x over previous
import functools

import jax
import jax.numpy as jnp
from jax import lax
from jax.experimental import pallas as pl
from jax.experimental.pallas import tpu as pltpu
from jax.experimental.pallas import tpu_sc as plsc

N = 10000
E = 320000
D = 128

NC = 2
NS = 16
NW = NC * NS
CH = 128
NCHUNKS = E // CH
BASE_CH = NCHUNKS // NW
REM_CH = NCHUNKS % NW

N_PAD = 10240
RPT = N_PAD // NS
LANES = 16

BR = 256
GRID = N_PAD // BR

_MESH = plsc.VectorSubcoreMesh(
    core_axis_name="c", subcore_axis_name="s", num_cores=NC, num_subcores=NS
)


def _sc_degree_body(ei, degp, dstv, degloc):
    c = lax.axis_index("c")
    s = lax.axis_index("s")
    w = s * NC + c

    zeros = jnp.zeros((LANES,), jnp.float32)

    def zinit(i, carry):
        degloc[pl.ds(i * LANES, LANES)] = zeros
        return carry

    lax.fori_loop(0, N_PAD // LANES, zinit, 0)

    nch = BASE_CH + jnp.where(w < REM_CH, 1, 0)
    ones = jnp.ones((LANES,), jnp.float32)

    def body(j, carry):
        off = (j * NW + w) * CH
        pltpu.sync_copy(ei.at[1, pl.ds(off, CH)], dstv)
        for k in range(CH // LANES):
            idx = dstv[pl.ds(k * LANES, LANES)]
            plsc.addupdate_scatter(degloc, [idx], ones)
        return carry

    lax.fori_loop(0, nch, body, 0)
    pltpu.sync_copy(degloc, degp.at[w])


def _sc_gather_scatter_body(hp, ei, zz, gp, srcv, dstv, rows, acc, sem):
    c = lax.axis_index("c")
    s = lax.axis_index("s")
    w = s * NC + c
    r0 = s * RPT
    pltpu.sync_copy(zz.at[pl.ds(r0, RPT), :], acc.at[pl.ds(r0, RPT), :])
    plsc.subcore_barrier()

    nch = BASE_CH + jnp.where(w < REM_CH, 1, 0)

    def body(j, carry):
        off = (j * NW + w) * CH
        pltpu.sync_copy(ei.at[0, pl.ds(off, CH)], srcv)
        pltpu.sync_copy(ei.at[1, pl.ds(off, CH)], dstv)
        pltpu.async_copy(hp.at[srcv], rows, sem).wait()
        pltpu.sync_copy(rows, acc.at[dstv], add=True)
        return carry

    lax.fori_loop(0, nch, body, 0)
    plsc.subcore_barrier()
    pltpu.sync_copy(acc.at[pl.ds(r0, RPT), :], gp.at[c, pl.ds(r0, RPT), :])


_DEG_SCRATCH = [
    pltpu.VMEM((CH,), jnp.int32),
    pltpu.VMEM((N_PAD,), jnp.float32),
]
_GS_SCRATCH = [
    pltpu.VMEM((CH,), jnp.int32),
    pltpu.VMEM((CH,), jnp.int32),
    pltpu.VMEM((CH, D), jnp.float32),
    pltpu.VMEM_SHARED((N_PAD, D), jnp.float32),
    pltpu.SemaphoreType.DMA,
]

_sc_degree = pl.kernel(
    _sc_degree_body,
    out_type=jax.ShapeDtypeStruct((NW, N_PAD), jnp.float32),
    mesh=_MESH,
    scratch_types=_DEG_SCRATCH,
    compiler_params=pltpu.CompilerParams(needs_layout_passes=False),
)

_sc_gather_scatter = pl.kernel(
    _sc_gather_scatter_body,
    out_type=jax.ShapeDtypeStruct((NC, N_PAD, D), jnp.float32),
    mesh=_MESH,
    scratch_types=_GS_SCRATCH,
)


def _tc1_body(degp_ref, x_ref, w1_ref, dinv_ref, hp1_ref):
    deg = jnp.sum(degp_ref[...], axis=0) + 1.0
    dinv = lax.rsqrt(deg)
    dinv_ref[...] = dinv
    hp1_ref[...] = dinv * jnp.dot(
        x_ref[...], w1_ref[...], preferred_element_type=jnp.float32
    )


def _tc2_body(dinv_ref, g_ref, hp1_ref, b1_ref, w2_ref, hp2_ref):
    dinv = dinv_ref[...]
    g = g_ref[0] + g_ref[1]
    h1 = jnp.maximum(dinv * (g + hp1_ref[...]) + b1_ref[...], 0.0)
    hp2_ref[...] = dinv * jnp.dot(
        h1, w2_ref[...], preferred_element_type=jnp.float32
    )


def _tc3_body(dinv_ref, g_ref, hp2_ref, b2_ref, out_ref):
    out_ref[...] = (
        dinv_ref[...] * (g_ref[0] + g_ref[1] + hp2_ref[...]) + b2_ref[...]
    )


_tc1 = pl.pallas_call(
    _tc1_body,
    grid=(GRID,),
    in_specs=[
        pl.BlockSpec((NW, BR, 1), lambda i: (0, i, 0)),
        pl.BlockSpec((BR, D), lambda i: (i, 0)),
        pl.BlockSpec((D, D), lambda i: (0, 0)),
    ],
    out_specs=[
        pl.BlockSpec((BR, 1), lambda i: (i, 0)),
        pl.BlockSpec((BR, D), lambda i: (i, 0)),
    ],
    out_shape=[
        jax.ShapeDtypeStruct((N_PAD, 1), jnp.float32),
        jax.ShapeDtypeStruct((N, D), jnp.float32),
    ],
)

_tc2 = pl.pallas_call(
    _tc2_body,
    grid=(GRID,),
    in_specs=[
        pl.BlockSpec((BR, 1), lambda i: (i, 0)),
        pl.BlockSpec((2, BR, D), lambda i: (0, i, 0)),
        pl.BlockSpec((BR, D), lambda i: (i, 0)),
        pl.BlockSpec((1, D), lambda i: (0, 0)),
        pl.BlockSpec((D, D), lambda i: (0, 0)),
    ],
    out_specs=pl.BlockSpec((BR, D), lambda i: (i, 0)),
    out_shape=jax.ShapeDtypeStruct((N, D), jnp.float32),
)

_tc3 = pl.pallas_call(
    _tc3_body,
    grid=(GRID,),
    in_specs=[
        pl.BlockSpec((BR, 1), lambda i: (i, 0)),
        pl.BlockSpec((2, BR, D), lambda i: (0, i, 0)),
        pl.BlockSpec((BR, D), lambda i: (i, 0)),
        pl.BlockSpec((1, D), lambda i: (0, 0)),
    ],
    out_specs=pl.BlockSpec((BR, D), lambda i: (i, 0)),
    out_shape=jax.ShapeDtypeStruct((N, D), jnp.float32),
)


def kernel(x, edge_index, W1, b1, W2, b2):
    ei = edge_index.astype(jnp.int32)
    zz = jnp.zeros((N_PAD, D), jnp.float32)

    degp = _sc_degree(ei)
    dinv, hp1 = _tc1(degp.reshape(NW, N_PAD, 1), x, W1)
    g1p = _sc_gather_scatter(hp1, ei, zz)
    hp2 = _tc2(dinv, g1p, hp1, b1.reshape(1, D), W2)
    g2p = _sc_gather_scatter(hp2, ei, zz)
    out = _tc3(dinv, g2p, hp2, b2.reshape(1, D))
    return out


# speedup vs baseline: 21.3094x; 1.6011x over previous
import functools

import jax
import jax.numpy as jnp
from jax import lax
from jax.experimental import pallas as pl
from jax.experimental.pallas import tpu as pltpu
from jax.experimental.pallas import tpu_sc as plsc

N = 10000
E = 320000
D = 128

NC = 2
NS = 16
NW = NC * NS
CH = 128
NCHUNKS = E // CH
BASE_CH = NCHUNKS // NW
REM_CH = NCHUNKS % NW

N_PAD = 10240
RPT = N_PAD // NS
LANES = 16

BR = 256
GRID = N_PAD // BR

_MESH = plsc.VectorSubcoreMesh(
    core_axis_name="c", subcore_axis_name="s", num_cores=NC, num_subcores=NS
)


def _sc_degree_body(ei, degp, dstv, degloc):
    c = lax.axis_index("c")
    s = lax.axis_index("s")
    w = s * NC + c

    zeros = jnp.zeros((LANES,), jnp.float32)

    def zinit(i, carry):
        degloc[pl.ds(i * LANES, LANES)] = zeros
        return carry

    lax.fori_loop(0, N_PAD // LANES, zinit, 0)

    nch = BASE_CH + jnp.where(w < REM_CH, 1, 0)
    ones = jnp.ones((LANES,), jnp.float32)

    def body(j, carry):
        off = (j * NW + w) * CH
        pltpu.sync_copy(ei.at[1, pl.ds(off, CH)], dstv)
        for k in range(CH // LANES):
            idx = dstv[pl.ds(k * LANES, LANES)]
            plsc.addupdate_scatter(degloc, [idx], ones)
        return carry

    lax.fori_loop(0, nch, body, 0)
    pltpu.sync_copy(degloc, degp.at[w])


def _sc_gather_scatter_body(hp, ei, zz, gp,
                            src0, src1, src2, src3,
                            dst0, dst1, dst2, dst3,
                            rows0, rows1, acc,
                            semg0, semg1, semi0, semi1, semi2, semi3):
    c = lax.axis_index("c")
    s = lax.axis_index("s")
    w = s * NC + c
    r0 = s * RPT
    nch = BASE_CH + jnp.where(w < REM_CH, 1, 0)

    pltpu.sync_copy(zz.at[pl.ds(r0, RPT), :], acc.at[pl.ds(r0, RPT), :])

    rowss = (rows0, rows1)
    semgs = (semg0, semg1)
    semis = (semi0, semi1, semi2, semi3)
    srcs = (src0, src1, src2, src3)
    dsts = (dst0, dst1, dst2, dst3)

    def fetch_idx(j, q):
        off = (j * NW + w) * CH
        pltpu.async_copy(ei.at[0, pl.ds(off, CH)], srcs[q], semis[q])
        pltpu.async_copy(ei.at[1, pl.ds(off, CH)], dsts[q], semis[q])

    def wait_idx(j, q):
        off = (j * NW + w) * CH
        pltpu.make_async_copy(ei.at[0, pl.ds(off, CH)], srcs[q],
                              semis[q]).wait()
        pltpu.make_async_copy(ei.at[1, pl.ds(off, CH)], dsts[q],
                              semis[q]).wait()

    def start_gather(b, q):
        pltpu.async_copy(hp.at[srcs[q]], rowss[b], semgs[b])

    def wait_gather(b, q):
        pltpu.make_async_copy(hp.at[srcs[q]], rowss[b], semgs[b]).wait()

    for q in range(4):
        fetch_idx(q, q)
    for b in range(2):
        wait_idx(b, b)
        start_gather(b, b)
    plsc.subcore_barrier()

    def quad(i, carry):
        for u in range(4):
            b, q = u % 2, u
            j = 4 * i + u

            @pl.when(j < nch)
            def _():
                wait_gather(b, q)
                pltpu.sync_copy(rowss[b], acc.at[dsts[q]], add=True)

                @pl.when(j + 2 < nch)
                def _():
                    wait_idx(j + 2, (q + 2) % 4)
                    start_gather(b, (q + 2) % 4)

                @pl.when(j + 4 < nch)
                def _():
                    fetch_idx(j + 4, q)

        return carry

    lax.fori_loop(0, (BASE_CH + 4) // 4, quad, 0)
    plsc.subcore_barrier()
    pltpu.sync_copy(acc.at[pl.ds(r0, RPT), :], gp.at[c, pl.ds(r0, RPT), :])


_DEG_SCRATCH = [
    pltpu.VMEM((CH,), jnp.int32),
    pltpu.VMEM((N_PAD,), jnp.float32),
]
_GS_SCRATCH = [
    pltpu.VMEM((CH,), jnp.int32),
    pltpu.VMEM((CH,), jnp.int32),
    pltpu.VMEM((CH,), jnp.int32),
    pltpu.VMEM((CH,), jnp.int32),
    pltpu.VMEM((CH,), jnp.int32),
    pltpu.VMEM((CH,), jnp.int32),
    pltpu.VMEM((CH,), jnp.int32),
    pltpu.VMEM((CH,), jnp.int32),
    pltpu.VMEM((CH, D), jnp.float32),
    pltpu.VMEM((CH, D), jnp.float32),
    pltpu.VMEM_SHARED((N_PAD, D), jnp.float32),
    pltpu.SemaphoreType.DMA,
    pltpu.SemaphoreType.DMA,
    pltpu.SemaphoreType.DMA,
    pltpu.SemaphoreType.DMA,
    pltpu.SemaphoreType.DMA,
    pltpu.SemaphoreType.DMA,
]

_sc_degree = pl.kernel(
    _sc_degree_body,
    out_type=jax.ShapeDtypeStruct((NW, N_PAD), jnp.float32),
    mesh=_MESH,
    scratch_types=_DEG_SCRATCH,
    compiler_params=pltpu.CompilerParams(needs_layout_passes=False),
)

_sc_gather_scatter = pl.kernel(
    _sc_gather_scatter_body,
    out_type=jax.ShapeDtypeStruct((NC, N_PAD, D), jnp.float32),
    mesh=_MESH,
    scratch_types=_GS_SCRATCH,
)


def _tc1_body(degp_ref, x_ref, w1_ref, dinv_ref, hp1_ref):
    deg = jnp.sum(degp_ref[...], axis=0) + 1.0
    dinv = lax.rsqrt(deg)
    dinv_ref[...] = dinv
    hp1_ref[...] = dinv * jnp.dot(
        x_ref[...], w1_ref[...], preferred_element_type=jnp.float32
    )


def _tc2_body(dinv_ref, g_ref, hp1_ref, b1_ref, w2_ref, hp2_ref):
    dinv = dinv_ref[...]
    g = g_ref[0] + g_ref[1]
    h1 = jnp.maximum(dinv * (g + hp1_ref[...]) + b1_ref[...], 0.0)
    hp2_ref[...] = dinv * jnp.dot(
        h1, w2_ref[...], preferred_element_type=jnp.float32
    )


def _tc3_body(dinv_ref, g_ref, hp2_ref, b2_ref, out_ref):
    out_ref[...] = (
        dinv_ref[...] * (g_ref[0] + g_ref[1] + hp2_ref[...]) + b2_ref[...]
    )


_tc1 = pl.pallas_call(
    _tc1_body,
    grid=(GRID,),
    in_specs=[
        pl.BlockSpec((NW, BR, 1), lambda i: (0, i, 0)),
        pl.BlockSpec((BR, D), lambda i: (i, 0)),
        pl.BlockSpec((D, D), lambda i: (0, 0)),
    ],
    out_specs=[
        pl.BlockSpec((BR, 1), lambda i: (i, 0)),
        pl.BlockSpec((BR, D), lambda i: (i, 0)),
    ],
    out_shape=[
        jax.ShapeDtypeStruct((N_PAD, 1), jnp.float32),
        jax.ShapeDtypeStruct((N, D), jnp.float32),
    ],
)

_tc2 = pl.pallas_call(
    _tc2_body,
    grid=(GRID,),
    in_specs=[
        pl.BlockSpec((BR, 1), lambda i: (i, 0)),
        pl.BlockSpec((2, BR, D), lambda i: (0, i, 0)),
        pl.BlockSpec((BR, D), lambda i: (i, 0)),
        pl.BlockSpec((1, D), lambda i: (0, 0)),
        pl.BlockSpec((D, D), lambda i: (0, 0)),
    ],
    out_specs=pl.BlockSpec((BR, D), lambda i: (i, 0)),
    out_shape=jax.ShapeDtypeStruct((N, D), jnp.float32),
)

_tc3 = pl.pallas_call(
    _tc3_body,
    grid=(GRID,),
    in_specs=[
        pl.BlockSpec((BR, 1), lambda i: (i, 0)),
        pl.BlockSpec((2, BR, D), lambda i: (0, i, 0)),
        pl.BlockSpec((BR, D), lambda i: (i, 0)),
        pl.BlockSpec((1, D), lambda i: (0, 0)),
    ],
    out_specs=pl.BlockSpec((BR, D), lambda i: (i, 0)),
    out_shape=jax.ShapeDtypeStruct((N, D), jnp.float32),
)


def kernel(x, edge_index, W1, b1, W2, b2):
    ei = edge_index.astype(jnp.int32)
    zz = jnp.zeros((N_PAD, D), jnp.float32)

    degp = _sc_degree(ei)
    dinv, hp1 = _tc1(degp.reshape(NW, N_PAD, 1), x, W1)
    g1p = _sc_gather_scatter(hp1, ei, zz)
    hp2 = _tc2(dinv, g1p, hp1, b1.reshape(1, D), W2)
    g2p = _sc_gather_scatter(hp2, ei, zz)
    out = _tc3(dinv, g2p, hp2, b2.reshape(1, D))
    return out


# speedup vs baseline: 29.9703x; 1.4064x over previous
import functools

import jax
import jax.numpy as jnp
from jax import lax
from jax.experimental import pallas as pl
from jax.experimental.pallas import tpu as pltpu
from jax.experimental.pallas import tpu_sc as plsc

N = 10000
E = 320000
D = 128

NC = 2
NS = 16
NW = NC * NS
CH = 128
NCHUNKS = E // CH
BASE_CH = NCHUNKS // NW
REM_CH = NCHUNKS % NW

N_PAD = 10240
RPT = N_PAD // NS
LANES = 16

BR = 256
GRID = N_PAD // BR

_MESH = plsc.VectorSubcoreMesh(
    core_axis_name="c", subcore_axis_name="s", num_cores=NC, num_subcores=NS
)


def _sc_degree_body(ei, degp, dst0, dst1, dst2, dst3, degloc,
                    semi0, semi1, semi2, semi3):
    c = lax.axis_index("c")
    s = lax.axis_index("s")
    w = s * NC + c

    dsts = (dst0, dst1, dst2, dst3)
    semis = (semi0, semi1, semi2, semi3)

    def fetch_idx(j, q):
        off = (j * NW + w) * CH
        pltpu.async_copy(ei.at[1, pl.ds(off, CH)], dsts[q], semis[q])

    def wait_idx(j, q):
        off = (j * NW + w) * CH
        pltpu.make_async_copy(ei.at[1, pl.ds(off, CH)], dsts[q],
                              semis[q]).wait()

    for q in range(4):
        fetch_idx(q, q)

    zeros = jnp.zeros((LANES,), jnp.float32)

    def zinit(i, carry):
        degloc[pl.ds(i * LANES, LANES)] = zeros
        return carry

    lax.fori_loop(0, N_PAD // LANES, zinit, 0)

    nch = BASE_CH + jnp.where(w < REM_CH, 1, 0)
    ones = jnp.ones((LANES,), jnp.float32)

    def quad(i, carry):
        for q in range(4):
            j = 4 * i + q

            @pl.when(j < nch)
            def _():
                wait_idx(j, q)
                for k in range(CH // LANES):
                    idx = dsts[q][pl.ds(k * LANES, LANES)]
                    plsc.addupdate_scatter(degloc, [idx], ones)

                @pl.when(j + 4 < nch)
                def _():
                    fetch_idx(j + 4, q)

        return carry

    lax.fori_loop(0, (BASE_CH + 4) // 4, quad, 0)
    pltpu.sync_copy(degloc, degp.at[w])


def _sc_gather_scatter_body(hp, ei, zz, gp,
                            src0, src1, src2, src3,
                            dst0, dst1, dst2, dst3,
                            rows0, rows1, acc,
                            semg0, semg1, semi0, semi1, semi2, semi3):
    c = lax.axis_index("c")
    s = lax.axis_index("s")
    w = s * NC + c
    r0 = s * RPT
    nch = BASE_CH + jnp.where(w < REM_CH, 1, 0)

    pltpu.sync_copy(zz.at[pl.ds(r0, RPT), :], acc.at[pl.ds(r0, RPT), :])

    rowss = (rows0, rows1)
    semgs = (semg0, semg1)
    semis = (semi0, semi1, semi2, semi3)
    srcs = (src0, src1, src2, src3)
    dsts = (dst0, dst1, dst2, dst3)

    def fetch_idx(j, q):
        off = (j * NW + w) * CH
        pltpu.async_copy(ei.at[0, pl.ds(off, CH)], srcs[q], semis[q])
        pltpu.async_copy(ei.at[1, pl.ds(off, CH)], dsts[q], semis[q])

    def wait_idx(j, q):
        off = (j * NW + w) * CH
        pltpu.make_async_copy(ei.at[0, pl.ds(off, CH)], srcs[q],
                              semis[q]).wait()
        pltpu.make_async_copy(ei.at[1, pl.ds(off, CH)], dsts[q],
                              semis[q]).wait()

    def start_gather(b, q):
        pltpu.async_copy(hp.at[srcs[q]], rowss[b], semgs[b])

    def wait_gather(b, q):
        pltpu.make_async_copy(hp.at[srcs[q]], rowss[b], semgs[b]).wait()

    for q in range(4):
        fetch_idx(q, q)
    for b in range(2):
        wait_idx(b, b)
        start_gather(b, b)
    plsc.subcore_barrier()

    def quad(i, carry):
        for u in range(4):
            b, q = u % 2, u
            j = 4 * i + u

            @pl.when(j < nch)
            def _():
                wait_gather(b, q)
                pltpu.sync_copy(rowss[b], acc.at[dsts[q]], add=True)

                @pl.when(j + 2 < nch)
                def _():
                    wait_idx(j + 2, (q + 2) % 4)
                    start_gather(b, (q + 2) % 4)

                @pl.when(j + 4 < nch)
                def _():
                    fetch_idx(j + 4, q)

        return carry

    lax.fori_loop(0, (BASE_CH + 4) // 4, quad, 0)
    plsc.subcore_barrier()
    pltpu.sync_copy(acc.at[pl.ds(r0, RPT), :], gp.at[c, pl.ds(r0, RPT), :])


_DEG_SCRATCH = [
    pltpu.VMEM((CH,), jnp.int32),
    pltpu.VMEM((CH,), jnp.int32),
    pltpu.VMEM((CH,), jnp.int32),
    pltpu.VMEM((CH,), jnp.int32),
    pltpu.VMEM((N_PAD,), jnp.float32),
    pltpu.SemaphoreType.DMA,
    pltpu.SemaphoreType.DMA,
    pltpu.SemaphoreType.DMA,
    pltpu.SemaphoreType.DMA,
]
_GS_SCRATCH = [
    pltpu.VMEM((CH,), jnp.int32),
    pltpu.VMEM((CH,), jnp.int32),
    pltpu.VMEM((CH,), jnp.int32),
    pltpu.VMEM((CH,), jnp.int32),
    pltpu.VMEM((CH,), jnp.int32),
    pltpu.VMEM((CH,), jnp.int32),
    pltpu.VMEM((CH,), jnp.int32),
    pltpu.VMEM((CH,), jnp.int32),
    pltpu.VMEM((CH, D), jnp.float32),
    pltpu.VMEM((CH, D), jnp.float32),
    pltpu.VMEM_SHARED((N_PAD, D), jnp.float32),
    pltpu.SemaphoreType.DMA,
    pltpu.SemaphoreType.DMA,
    pltpu.SemaphoreType.DMA,
    pltpu.SemaphoreType.DMA,
    pltpu.SemaphoreType.DMA,
    pltpu.SemaphoreType.DMA,
]

_sc_degree = pl.kernel(
    _sc_degree_body,
    out_type=jax.ShapeDtypeStruct((NW, N_PAD), jnp.float32),
    mesh=_MESH,
    scratch_types=_DEG_SCRATCH,
    compiler_params=pltpu.CompilerParams(needs_layout_passes=False),
)

_sc_gather_scatter = pl.kernel(
    _sc_gather_scatter_body,
    out_type=jax.ShapeDtypeStruct((NC, N_PAD, D), jnp.float32),
    mesh=_MESH,
    scratch_types=_GS_SCRATCH,
)


def _dinv_col(degp_blk):
    deg = jnp.sum(degp_blk, axis=0) + 1.0
    return lax.rsqrt(deg)[:, None]


def _tc_mm1_body(x_ref, w1_ref, p1_ref):
    p1_ref[...] = jnp.dot(
        x_ref[...], w1_ref[...], preferred_element_type=jnp.float32
    )


def _tc_scale1_body(degp_ref, p1_ref, hp1_ref):
    hp1_ref[...] = _dinv_col(degp_ref[...]) * p1_ref[...]


def _tc2_body(degp_ref, g_ref, hp1_ref, b1_ref, w2_ref, hp2_ref):
    dinv = _dinv_col(degp_ref[...])
    g = g_ref[0] + g_ref[1]
    h1 = jnp.maximum(dinv * (g + hp1_ref[...]) + b1_ref[...], 0.0)
    hp2_ref[...] = dinv * jnp.dot(
        h1, w2_ref[...], preferred_element_type=jnp.float32
    )


def _tc3_body(degp_ref, g_ref, hp2_ref, b2_ref, out_ref):
    out_ref[...] = (
        _dinv_col(degp_ref[...]) * (g_ref[0] + g_ref[1] + hp2_ref[...])
        + b2_ref[...]
    )


_tc_mm1 = pl.pallas_call(
    _tc_mm1_body,
    grid=(GRID,),
    in_specs=[
        pl.BlockSpec((BR, D), lambda i: (i, 0)),
        pl.BlockSpec((D, D), lambda i: (0, 0)),
    ],
    out_specs=pl.BlockSpec((BR, D), lambda i: (i, 0)),
    out_shape=jax.ShapeDtypeStruct((N, D), jnp.float32),
)

_tc_scale1 = pl.pallas_call(
    _tc_scale1_body,
    grid=(GRID,),
    in_specs=[
        pl.BlockSpec((NW, BR), lambda i: (0, i)),
        pl.BlockSpec((BR, D), lambda i: (i, 0)),
    ],
    out_specs=pl.BlockSpec((BR, D), lambda i: (i, 0)),
    out_shape=jax.ShapeDtypeStruct((N, D), jnp.float32),
)

_tc2 = pl.pallas_call(
    _tc2_body,
    grid=(GRID,),
    in_specs=[
        pl.BlockSpec((NW, BR), lambda i: (0, i)),
        pl.BlockSpec((2, BR, D), lambda i: (0, i, 0)),
        pl.BlockSpec((BR, D), lambda i: (i, 0)),
        pl.BlockSpec((1, D), lambda i: (0, 0)),
        pl.BlockSpec((D, D), lambda i: (0, 0)),
    ],
    out_specs=pl.BlockSpec((BR, D), lambda i: (i, 0)),
    out_shape=jax.ShapeDtypeStruct((N, D), jnp.float32),
)

_tc3 = pl.pallas_call(
    _tc3_body,
    grid=(GRID,),
    in_specs=[
        pl.BlockSpec((NW, BR), lambda i: (0, i)),
        pl.BlockSpec((2, BR, D), lambda i: (0, i, 0)),
        pl.BlockSpec((BR, D), lambda i: (i, 0)),
        pl.BlockSpec((1, D), lambda i: (0, 0)),
    ],
    out_specs=pl.BlockSpec((BR, D), lambda i: (i, 0)),
    out_shape=jax.ShapeDtypeStruct((N, D), jnp.float32),
)


def kernel(x, edge_index, W1, b1, W2, b2):
    ei = edge_index.astype(jnp.int32)
    zz = jnp.zeros((N_PAD, D), jnp.float32)

    degp = _sc_degree(ei)
    p1 = _tc_mm1(x, W1)
    hp1 = _tc_scale1(degp, p1)
    g1p = _sc_gather_scatter(hp1, ei, zz)
    hp2 = _tc2(degp, g1p, hp1, b1.reshape(1, D), W2)
    g2p = _sc_gather_scatter(hp2, ei, zz)
    out = _tc3(degp, g2p, hp2, b2.reshape(1, D))
    return out


# speedup vs baseline: 35.5905x; 1.1875x over previous
import functools

import jax
import jax.numpy as jnp
from jax import lax
from jax.experimental import pallas as pl
from jax.experimental.pallas import tpu as pltpu
from jax.experimental.pallas import tpu_sc as plsc

N = 10000
E = 320000
D = 128

NC = 2
NS = 16
NW = NC * NS
CH = 128
NCHUNKS = E // CH
BASE_CH = NCHUNKS // NW
REM_CH = NCHUNKS % NW

N_PAD = 10240
RPT = 640
RPT_LAST = N - 15 * RPT
LANES = 16

BR = 256
GRID = N_PAD // BR
BR2 = 512
GRID2 = N_PAD // BR2
BRE = 1024
GRIDE = N_PAD // BRE

_MESH = plsc.VectorSubcoreMesh(
    core_axis_name="c", subcore_axis_name="s", num_cores=NC, num_subcores=NS
)


def _sc_degree_body(ei, degp, dst0, dst1, dst2, dst3, degloc,
                    semi0, semi1, semi2, semi3):
    c = lax.axis_index("c")
    s = lax.axis_index("s")
    w = s * NC + c

    dsts = (dst0, dst1, dst2, dst3)
    semis = (semi0, semi1, semi2, semi3)

    def fetch_idx(j, q):
        off = (j * NW + w) * CH
        pltpu.async_copy(ei.at[1, pl.ds(off, CH)], dsts[q], semis[q])

    def wait_idx(j, q):
        off = (j * NW + w) * CH
        pltpu.make_async_copy(ei.at[1, pl.ds(off, CH)], dsts[q],
                              semis[q]).wait()

    for q in range(4):
        fetch_idx(q, q)

    zeros = jnp.zeros((LANES,), jnp.float32)

    def zinit(i, carry):
        degloc[pl.ds(i * LANES, LANES)] = zeros
        return carry

    lax.fori_loop(0, N_PAD // LANES, zinit, 0)

    nch = BASE_CH + jnp.where(w < REM_CH, 1, 0)
    ones = jnp.ones((LANES,), jnp.float32)

    def quad(i, carry):
        for q in range(4):
            j = 4 * i + q

            @pl.when(j < nch)
            def _():
                wait_idx(j, q)
                for k in range(CH // LANES):
                    idx = dsts[q][pl.ds(k * LANES, LANES)]
                    plsc.addupdate_scatter(degloc, [idx], ones)

                @pl.when(j + 4 < nch)
                def _():
                    fetch_idx(j + 4, q)

        return carry

    lax.fori_loop(0, (BASE_CH + 4) // 4, quad, 0)
    pltpu.sync_copy(degloc, degp.at[w])


def _sc_gather_scatter_body(hp, ei, zz, gp,
                            src0, src1, src2, src3,
                            dst0, dst1, dst2, dst3,
                            rows0, rows1, rows2, acc,
                            semg0, semg1, semg2,
                            sems0, sems1, sems2,
                            semi0, semi1, semi2, semi3):
    c = lax.axis_index("c")
    s = lax.axis_index("s")
    w = s * NC + c
    r0 = s * RPT
    nch = BASE_CH + jnp.where(w < REM_CH, 1, 0)

    @pl.when(s < NS - 1)
    def _():
        pltpu.sync_copy(zz.at[pl.ds(r0, RPT), :], acc.at[pl.ds(r0, RPT), :])

    @pl.when(s == NS - 1)
    def _():
        pltpu.sync_copy(zz.at[pl.ds(r0, RPT_LAST), :],
                        acc.at[pl.ds(r0, RPT_LAST), :])

    rowss = (rows0, rows1, rows2)
    semgs = (semg0, semg1, semg2)
    semss = (sems0, sems1, sems2)
    semis = (semi0, semi1, semi2, semi3)
    srcs = (src0, src1, src2, src3)
    dsts = (dst0, dst1, dst2, dst3)

    def fetch_idx(j, q4):
        off = (j * NW + w) * CH
        pltpu.async_copy(ei.at[0, pl.ds(off, CH)], srcs[q4], semis[q4])
        pltpu.async_copy(ei.at[1, pl.ds(off, CH)], dsts[q4], semis[q4])

    def wait_idx(j, q4):
        off = (j * NW + w) * CH
        pltpu.make_async_copy(ei.at[0, pl.ds(off, CH)], srcs[q4],
                              semis[q4]).wait()
        pltpu.make_async_copy(ei.at[1, pl.ds(off, CH)], dsts[q4],
                              semis[q4]).wait()

    def start_gather(q4, q3):
        pltpu.async_copy(hp.at[srcs[q4]], rowss[q3], semgs[q3])

    def wait_gather(q4, q3):
        pltpu.make_async_copy(hp.at[srcs[q4]], rowss[q3], semgs[q3]).wait()

    def start_scatter(q4, q3):
        pltpu.async_copy(rowss[q3], acc.at[dsts[q4]], semss[q3], add=True)

    def wait_scatter(q4, q3):
        pltpu.make_async_copy(rowss[q3], acc.at[dsts[q4]],
                              semss[q3]).wait()

    for q in range(4):
        fetch_idx(q, q)
    for q in range(2):
        wait_idx(q, q)
        start_gather(q, q)
    plsc.subcore_barrier()

    def twelve(i, carry):
        for u in range(12):
            j = 12 * i + u

            @pl.when(j < nch)
            def _():
                wait_gather(u % 4, u % 3)
                start_scatter(u % 4, u % 3)

                @pl.when(j + 2 < nch)
                def _():
                    @pl.when(j >= 1)
                    def _():
                        wait_scatter((u + 3) % 4, (u + 2) % 3)

                        @pl.when(j + 3 < nch)
                        def _():
                            fetch_idx(j + 3, (u + 3) % 4)

                    wait_idx(j + 2, (u + 2) % 4)
                    start_gather((u + 2) % 4, (u + 2) % 3)

        return carry

    lax.fori_loop(0, (BASE_CH + 12) // 12, twelve, 0)

    for q in range(3):
        wait_scatter(q, q)

    plsc.subcore_barrier()

    @pl.when(s < NS - 1)
    def _():
        pltpu.sync_copy(acc.at[pl.ds(r0, RPT), :],
                        gp.at[c, pl.ds(r0, RPT), :])

    @pl.when(s == NS - 1)
    def _():
        pltpu.sync_copy(acc.at[pl.ds(r0, RPT_LAST), :],
                        gp.at[c, pl.ds(r0, RPT_LAST), :])


_DEG_SCRATCH = [
    pltpu.VMEM((CH,), jnp.int32),
    pltpu.VMEM((CH,), jnp.int32),
    pltpu.VMEM((CH,), jnp.int32),
    pltpu.VMEM((CH,), jnp.int32),
    pltpu.VMEM((N_PAD,), jnp.float32),
    pltpu.SemaphoreType.DMA,
    pltpu.SemaphoreType.DMA,
    pltpu.SemaphoreType.DMA,
    pltpu.SemaphoreType.DMA,
]
_GS_SCRATCH = [
    pltpu.VMEM((CH,), jnp.int32),
    pltpu.VMEM((CH,), jnp.int32),
    pltpu.VMEM((CH,), jnp.int32),
    pltpu.VMEM((CH,), jnp.int32),
    pltpu.VMEM((CH,), jnp.int32),
    pltpu.VMEM((CH,), jnp.int32),
    pltpu.VMEM((CH,), jnp.int32),
    pltpu.VMEM((CH,), jnp.int32),
    pltpu.VMEM((CH, D), jnp.float32),
    pltpu.VMEM((CH, D), jnp.float32),
    pltpu.VMEM((CH, D), jnp.float32),
    pltpu.VMEM_SHARED((N, D), jnp.float32),
] + [pltpu.SemaphoreType.DMA] * 10

_sc_degree = pl.kernel(
    _sc_degree_body,
    out_type=jax.ShapeDtypeStruct((NW, N_PAD), jnp.float32),
    mesh=_MESH,
    scratch_types=_DEG_SCRATCH,
    compiler_params=pltpu.CompilerParams(needs_layout_passes=False),
)

_sc_gather_scatter = pl.kernel(
    _sc_gather_scatter_body,
    out_type=jax.ShapeDtypeStruct((NC, N, D), jnp.float32),
    mesh=_MESH,
    scratch_types=_GS_SCRATCH,
)


def _dinv_col(degp_blk):
    deg = jnp.sum(degp_blk, axis=0) + 1.0
    return lax.rsqrt(deg)[:, None]


def _tc_mm1_body(x_ref, w1_ref, p1_ref):
    p1_ref[...] = jnp.dot(
        x_ref[...], w1_ref[...], preferred_element_type=jnp.float32
    )


def _tc_scale1_body(degp_ref, p1_ref, hp1_ref):
    hp1_ref[...] = _dinv_col(degp_ref[...]) * p1_ref[...]


def _tc2_body(degp_ref, g_ref, hp1_ref, b1_ref, w2_ref, hp2_ref):
    dinv = _dinv_col(degp_ref[...])
    g = g_ref[0] + g_ref[1]
    h1 = jnp.maximum(dinv * (g + hp1_ref[...]) + b1_ref[...], 0.0)
    hp2_ref[...] = dinv * jnp.dot(
        h1, w2_ref[...], preferred_element_type=jnp.float32
    )


def _tc3_body(degp_ref, g_ref, hp2_ref, b2_ref, out_ref):
    out_ref[...] = (
        _dinv_col(degp_ref[...]) * (g_ref[0] + g_ref[1] + hp2_ref[...])
        + b2_ref[...]
    )


_tc_mm1 = pl.pallas_call(
    _tc_mm1_body,
    grid=(GRID,),
    in_specs=[
        pl.BlockSpec((BR, D), lambda i: (i, 0)),
        pl.BlockSpec((D, D), lambda i: (0, 0)),
    ],
    out_specs=pl.BlockSpec((BR, D), lambda i: (i, 0)),
    out_shape=jax.ShapeDtypeStruct((N, D), jnp.float32),
)

_tc_scale1 = pl.pallas_call(
    _tc_scale1_body,
    grid=(GRIDE,),
    in_specs=[
        pl.BlockSpec((NW, BRE), lambda i: (0, i)),
        pl.BlockSpec((BRE, D), lambda i: (i, 0)),
    ],
    out_specs=pl.BlockSpec((BRE, D), lambda i: (i, 0)),
    out_shape=jax.ShapeDtypeStruct((N, D), jnp.float32),
)

_tc2 = pl.pallas_call(
    _tc2_body,
    grid=(GRID2,),
    in_specs=[
        pl.BlockSpec((NW, BR2), lambda i: (0, i)),
        pl.BlockSpec((2, BR2, D), lambda i: (0, i, 0)),
        pl.BlockSpec((BR2, D), lambda i: (i, 0)),
        pl.BlockSpec((1, D), lambda i: (0, 0)),
        pl.BlockSpec((D, D), lambda i: (0, 0)),
    ],
    out_specs=pl.BlockSpec((BR2, D), lambda i: (i, 0)),
    out_shape=jax.ShapeDtypeStruct((N, D), jnp.float32),
)

_tc3 = pl.pallas_call(
    _tc3_body,
    grid=(GRID2,),
    in_specs=[
        pl.BlockSpec((NW, BR2), lambda i: (0, i)),
        pl.BlockSpec((2, BR2, D), lambda i: (0, i, 0)),
        pl.BlockSpec((BR2, D), lambda i: (i, 0)),
        pl.BlockSpec((1, D), lambda i: (0, 0)),
    ],
    out_specs=pl.BlockSpec((BR2, D), lambda i: (i, 0)),
    out_shape=jax.ShapeDtypeStruct((N, D), jnp.float32),
)


def kernel(x, edge_index, W1, b1, W2, b2):
    ei = edge_index.astype(jnp.int32)
    zz = jnp.zeros((N, D), jnp.float32)

    degp = _sc_degree(ei)
    p1 = _tc_mm1(x, W1)
    hp1 = _tc_scale1(degp, p1)
    g1p = _sc_gather_scatter(hp1, ei, zz)
    hp2 = _tc2(degp, g1p, hp1, b1.reshape(1, D), W2)
    g2p = _sc_gather_scatter(hp2, ei, zz)
    out = _tc3(degp, g2p, hp2, b2.reshape(1, D))
    return out


# speedup vs baseline: 37.0132x; 1.0400x over previous
import functools

import jax
import jax.numpy as jnp
from jax import lax
from jax.experimental import pallas as pl
from jax.experimental.pallas import tpu as pltpu
from jax.experimental.pallas import tpu_sc as plsc

N = 10000
E = 320000
D = 128

NC = 2
NS = 16
NW = NC * NS
CH = 128
NCHUNKS = E // CH
BASE_CH = NCHUNKS // NW
REM_CH = NCHUNKS % NW

N_PAD = 10240
RPT = 640
RPT_LAST = N - 15 * RPT
LANES = 16

BR = 256
GRID = N_PAD // BR
BR2 = 1024
GRID2 = N_PAD // BR2
BRE = 1024
GRIDE = N_PAD // BRE

_MESH = plsc.VectorSubcoreMesh(
    core_axis_name="c", subcore_axis_name="s", num_cores=NC, num_subcores=NS
)


def _sc_degree_body(ei, degp, dst0, dst1, dst2, dst3, degloc,
                    semi0, semi1, semi2, semi3):
    c = lax.axis_index("c")
    s = lax.axis_index("s")
    w = s * NC + c

    dsts = (dst0, dst1, dst2, dst3)
    semis = (semi0, semi1, semi2, semi3)

    def fetch_idx(j, q):
        off = (j * NW + w) * CH
        pltpu.async_copy(ei.at[1, pl.ds(off, CH)], dsts[q], semis[q])

    def wait_idx(j, q):
        off = (j * NW + w) * CH
        pltpu.make_async_copy(ei.at[1, pl.ds(off, CH)], dsts[q],
                              semis[q]).wait()

    for q in range(4):
        fetch_idx(q, q)

    zeros = jnp.zeros((LANES,), jnp.float32)

    def zinit(i, carry):
        degloc[pl.ds(i * LANES, LANES)] = zeros
        return carry

    lax.fori_loop(0, N_PAD // LANES, zinit, 0)

    nch = BASE_CH + jnp.where(w < REM_CH, 1, 0)
    ones = jnp.ones((LANES,), jnp.float32)

    def quad(i, carry):
        for q in range(4):
            j = 4 * i + q

            @pl.when(j < nch)
            def _():
                wait_idx(j, q)
                for k in range(CH // LANES):
                    idx = dsts[q][pl.ds(k * LANES, LANES)]
                    plsc.addupdate_scatter(degloc, [idx], ones)

                @pl.when(j + 4 < nch)
                def _():
                    fetch_idx(j + 4, q)

        return carry

    lax.fori_loop(0, (BASE_CH + 4) // 4, quad, 0)
    pltpu.sync_copy(degloc, degp.at[w])


def _sc_gather_scatter_body(hp, ei, zz, gp,
                            src0, src1, src2, src3,
                            dst0, dst1, dst2, dst3,
                            rows0, rows1, rows2, acc,
                            semg0, semg1, semg2,
                            sems0, sems1, sems2,
                            semi0, semi1, semi2, semi3):
    c = lax.axis_index("c")
    s = lax.axis_index("s")
    w = s * NC + c
    r0 = s * RPT
    nch = BASE_CH + jnp.where(w < REM_CH, 1, 0)

    @pl.when(s < NS - 1)
    def _():
        pltpu.sync_copy(zz.at[pl.ds(r0, RPT), :], acc.at[pl.ds(r0, RPT), :])

    @pl.when(s == NS - 1)
    def _():
        pltpu.sync_copy(zz.at[pl.ds(r0, RPT_LAST), :],
                        acc.at[pl.ds(r0, RPT_LAST), :])

    rowss = (rows0, rows1, rows2)
    semgs = (semg0, semg1, semg2)
    semss = (sems0, sems1, sems2)
    semis = (semi0, semi1, semi2, semi3)
    srcs = (src0, src1, src2, src3)
    dsts = (dst0, dst1, dst2, dst3)

    def fetch_idx(j, q4):
        off = (j * NW + w) * CH
        pltpu.async_copy(ei.at[0, pl.ds(off, CH)], srcs[q4], semis[q4])
        pltpu.async_copy(ei.at[1, pl.ds(off, CH)], dsts[q4], semis[q4])

    def wait_idx(j, q4):
        off = (j * NW + w) * CH
        pltpu.make_async_copy(ei.at[0, pl.ds(off, CH)], srcs[q4],
                              semis[q4]).wait()
        pltpu.make_async_copy(ei.at[1, pl.ds(off, CH)], dsts[q4],
                              semis[q4]).wait()

    def start_gather(q4, q3):
        pltpu.async_copy(hp.at[srcs[q4]], rowss[q3], semgs[q3])

    def wait_gather(q4, q3):
        pltpu.make_async_copy(hp.at[srcs[q4]], rowss[q3], semgs[q3]).wait()

    def start_scatter(q4, q3):
        pltpu.async_copy(rowss[q3], acc.at[dsts[q4]], semss[q3], add=True)

    def wait_scatter(q4, q3):
        pltpu.make_async_copy(rowss[q3], acc.at[dsts[q4]],
                              semss[q3]).wait()

    for q in range(4):
        fetch_idx(q, q)
    for q in range(2):
        wait_idx(q, q)
        start_gather(q, q)
    plsc.subcore_barrier()

    def twelve(i, carry):
        for u in range(12):
            j = 12 * i + u

            @pl.when(j < nch)
            def _():
                wait_gather(u % 4, u % 3)
                start_scatter(u % 4, u % 3)

                @pl.when(j + 2 < nch)
                def _():
                    @pl.when(j >= 1)
                    def _():
                        wait_scatter((u + 3) % 4, (u + 2) % 3)

                        @pl.when(j + 3 < nch)
                        def _():
                            fetch_idx(j + 3, (u + 3) % 4)

                    wait_idx(j + 2, (u + 2) % 4)
                    start_gather((u + 2) % 4, (u + 2) % 3)

        return carry

    lax.fori_loop(0, (BASE_CH + 12) // 12, twelve, 0)

    for q in range(3):
        wait_scatter(q, q)

    plsc.subcore_barrier()

    @pl.when(s < NS - 1)
    def _():
        pltpu.sync_copy(acc.at[pl.ds(r0, RPT), :],
                        gp.at[c, pl.ds(r0, RPT), :])

    @pl.when(s == NS - 1)
    def _():
        pltpu.sync_copy(acc.at[pl.ds(r0, RPT_LAST), :],
                        gp.at[c, pl.ds(r0, RPT_LAST), :])


_DEG_SCRATCH = [
    pltpu.VMEM((CH,), jnp.int32),
    pltpu.VMEM((CH,), jnp.int32),
    pltpu.VMEM((CH,), jnp.int32),
    pltpu.VMEM((CH,), jnp.int32),
    pltpu.VMEM((N_PAD,), jnp.float32),
    pltpu.SemaphoreType.DMA,
    pltpu.SemaphoreType.DMA,
    pltpu.SemaphoreType.DMA,
    pltpu.SemaphoreType.DMA,
]
_GS_SCRATCH = [
    pltpu.VMEM((CH,), jnp.int32),
    pltpu.VMEM((CH,), jnp.int32),
    pltpu.VMEM((CH,), jnp.int32),
    pltpu.VMEM((CH,), jnp.int32),
    pltpu.VMEM((CH,), jnp.int32),
    pltpu.VMEM((CH,), jnp.int32),
    pltpu.VMEM((CH,), jnp.int32),
    pltpu.VMEM((CH,), jnp.int32),
    pltpu.VMEM((CH, D), jnp.float32),
    pltpu.VMEM((CH, D), jnp.float32),
    pltpu.VMEM((CH, D), jnp.float32),
    pltpu.VMEM_SHARED((N, D), jnp.float32),
] + [pltpu.SemaphoreType.DMA] * 10

_sc_degree = pl.kernel(
    _sc_degree_body,
    out_type=jax.ShapeDtypeStruct((NW, N_PAD), jnp.float32),
    mesh=_MESH,
    scratch_types=_DEG_SCRATCH,
    compiler_params=pltpu.CompilerParams(needs_layout_passes=False),
)

_sc_gather_scatter = pl.kernel(
    _sc_gather_scatter_body,
    out_type=jax.ShapeDtypeStruct((NC, N, D), jnp.float32),
    mesh=_MESH,
    scratch_types=_GS_SCRATCH,
)


def _dinv_col(degp_blk):
    deg = jnp.sum(degp_blk, axis=0) + 1.0
    return lax.rsqrt(deg)[:, None]


def _tc_mm1_body(x_ref, w1_ref, p1_ref):
    p1_ref[...] = jnp.dot(
        x_ref[...], w1_ref[...], preferred_element_type=jnp.float32
    )


def _tc_scale1_body(degp_ref, p1_ref, hp1_ref):
    hp1_ref[...] = _dinv_col(degp_ref[...]) * p1_ref[...]


def _tc2_body(degp_ref, g_ref, hp1_ref, b1_ref, w2_ref, hp2_ref):
    dinv = _dinv_col(degp_ref[...])
    g = g_ref[0] + g_ref[1]
    h1 = jnp.maximum(dinv * (g + hp1_ref[...]) + b1_ref[...], 0.0)
    hp2_ref[...] = dinv * jnp.dot(
        h1, w2_ref[...], preferred_element_type=jnp.float32
    )


def _tc3_body(degp_ref, g_ref, hp2_ref, b2_ref, out_ref):
    out_ref[...] = (
        _dinv_col(degp_ref[...]) * (g_ref[0] + g_ref[1] + hp2_ref[...])
        + b2_ref[...]
    )


_tc_mm1 = pl.pallas_call(
    _tc_mm1_body,
    grid=(GRID,),
    in_specs=[
        pl.BlockSpec((BR, D), lambda i: (i, 0)),
        pl.BlockSpec((D, D), lambda i: (0, 0)),
    ],
    out_specs=pl.BlockSpec((BR, D), lambda i: (i, 0)),
    out_shape=jax.ShapeDtypeStruct((N, D), jnp.float32),
)

_tc_scale1 = pl.pallas_call(
    _tc_scale1_body,
    grid=(GRIDE,),
    in_specs=[
        pl.BlockSpec((NW, BRE), lambda i: (0, i)),
        pl.BlockSpec((BRE, D), lambda i: (i, 0)),
    ],
    out_specs=pl.BlockSpec((BRE, D), lambda i: (i, 0)),
    out_shape=jax.ShapeDtypeStruct((N, D), jnp.float32),
)

_tc2 = pl.pallas_call(
    _tc2_body,
    grid=(GRID2,),
    in_specs=[
        pl.BlockSpec((NW, BR2), lambda i: (0, i)),
        pl.BlockSpec((2, BR2, D), lambda i: (0, i, 0)),
        pl.BlockSpec((BR2, D), lambda i: (i, 0)),
        pl.BlockSpec((1, D), lambda i: (0, 0)),
        pl.BlockSpec((D, D), lambda i: (0, 0)),
    ],
    out_specs=pl.BlockSpec((BR2, D), lambda i: (i, 0)),
    out_shape=jax.ShapeDtypeStruct((N, D), jnp.float32),
)

_tc3 = pl.pallas_call(
    _tc3_body,
    grid=(GRID2,),
    in_specs=[
        pl.BlockSpec((NW, BR2), lambda i: (0, i)),
        pl.BlockSpec((2, BR2, D), lambda i: (0, i, 0)),
        pl.BlockSpec((BR2, D), lambda i: (i, 0)),
        pl.BlockSpec((1, D), lambda i: (0, 0)),
    ],
    out_specs=pl.BlockSpec((BR2, D), lambda i: (i, 0)),
    out_shape=jax.ShapeDtypeStruct((N, D), jnp.float32),
)


def kernel(x, edge_index, W1, b1, W2, b2):
    ei = edge_index.astype(jnp.int32)
    zz = jnp.zeros((N, D), jnp.float32)

    degp = _sc_degree(ei)
    p1 = _tc_mm1(x, W1)
    hp1 = _tc_scale1(degp, p1)
    g1p = _sc_gather_scatter(hp1, ei, zz)
    hp2 = _tc2(degp, g1p, hp1, b1.reshape(1, D), W2)
    g2p = _sc_gather_scatter(hp2, ei, zz)
    out = _tc3(degp, g2p, hp2, b2.reshape(1, D))
    return out


# speedup vs baseline: 38.0593x; 1.0283x over previous
import functools

import jax
import jax.numpy as jnp
from jax import lax
from jax.experimental import pallas as pl
from jax.experimental.pallas import tpu as pltpu
from jax.experimental.pallas import tpu_sc as plsc

N = 10000
E = 320000
D = 128

NC = 2
NS = 16
NW = NC * NS
CH = 128
NCHUNKS = E // CH
BASE_CH = NCHUNKS // NW
REM_CH = NCHUNKS % NW

N_PAD = 10240
RPT = 640
RPT_LAST = N - 15 * RPT
LANES = 16

BR = 256
GRID = N_PAD // BR
BR2 = 1024
GRID2 = N_PAD // BR2
BRE = 1024
GRIDE = N_PAD // BRE

_MESH = plsc.VectorSubcoreMesh(
    core_axis_name="c", subcore_axis_name="s", num_cores=NC, num_subcores=NS
)


def _sc_degree_body(ei, degp, dst0, dst1, dst2, dst3, degloc,
                    semi0, semi1, semi2, semi3):
    c = lax.axis_index("c")
    s = lax.axis_index("s")
    w = s * NC + c

    dsts = (dst0, dst1, dst2, dst3)
    semis = (semi0, semi1, semi2, semi3)

    def fetch_idx(j, q):
        off = (j * NW + w) * CH
        pltpu.async_copy(ei.at[1, pl.ds(off, CH)], dsts[q], semis[q])

    def wait_idx(j, q):
        off = (j * NW + w) * CH
        pltpu.make_async_copy(ei.at[1, pl.ds(off, CH)], dsts[q],
                              semis[q]).wait()

    for q in range(4):
        fetch_idx(q, q)

    zeros = jnp.zeros((LANES,), jnp.float32)

    def zinit(i, carry):
        degloc[pl.ds(i * LANES, LANES)] = zeros
        return carry

    lax.fori_loop(0, N_PAD // LANES, zinit, 0)

    nch = BASE_CH + jnp.where(w < REM_CH, 1, 0)
    ones = jnp.ones((LANES,), jnp.float32)

    def quad(i, carry):
        for q in range(4):
            j = 4 * i + q

            @pl.when(j < nch)
            def _():
                wait_idx(j, q)
                for k in range(CH // LANES):
                    idx = dsts[q][pl.ds(k * LANES, LANES)]
                    plsc.addupdate_scatter(degloc, [idx], ones)

                @pl.when(j + 4 < nch)
                def _():
                    fetch_idx(j + 4, q)

        return carry

    lax.fori_loop(0, (BASE_CH + 4) // 4, quad, 0)
    pltpu.sync_copy(degloc, degp.at[w])


def _sc_gather_scatter_body(hp, ei, zz, gp,
                            src0, src1, src2, src3,
                            dst0, dst1, dst2, dst3,
                            rows0, rows1, rows2, acc,
                            semg0, semg1, semg2,
                            sems0, sems1, sems2,
                            semi0, semi1, semi2, semi3):
    c = lax.axis_index("c")
    s = lax.axis_index("s")
    w = s * NC + c
    r0 = s * RPT
    nch = BASE_CH + jnp.where(w < REM_CH, 1, 0)

    @pl.when(s < NS - 1)
    def _():
        pltpu.sync_copy(zz.at[pl.ds(r0, RPT), :], acc.at[pl.ds(r0, RPT), :])

    @pl.when(s == NS - 1)
    def _():
        pltpu.sync_copy(zz.at[pl.ds(r0, RPT_LAST), :],
                        acc.at[pl.ds(r0, RPT_LAST), :])

    rowss = (rows0, rows1, rows2)
    semgs = (semg0, semg1, semg2)
    semss = (sems0, sems1, sems2)
    semis = (semi0, semi1, semi2, semi3)
    srcs = (src0, src1, src2, src3)
    dsts = (dst0, dst1, dst2, dst3)

    def fetch_idx(j, q4):
        off = (j * NW + w) * CH
        pltpu.async_copy(ei.at[0, pl.ds(off, CH)], srcs[q4], semis[q4])
        pltpu.async_copy(ei.at[1, pl.ds(off, CH)], dsts[q4], semis[q4])

    def wait_idx(j, q4):
        off = (j * NW + w) * CH
        pltpu.make_async_copy(ei.at[0, pl.ds(off, CH)], srcs[q4],
                              semis[q4]).wait()
        pltpu.make_async_copy(ei.at[1, pl.ds(off, CH)], dsts[q4],
                              semis[q4]).wait()

    def start_gather(q4, q3):
        pltpu.async_copy(hp.at[srcs[q4]], rowss[q3], semgs[q3])

    def wait_gather(q4, q3):
        pltpu.make_async_copy(hp.at[srcs[q4]], rowss[q3], semgs[q3]).wait()

    def start_scatter(q4, q3):
        pltpu.async_copy(rowss[q3], acc.at[dsts[q4]], semss[q3], add=True)

    def wait_scatter(q4, q3):
        pltpu.make_async_copy(rowss[q3], acc.at[dsts[q4]],
                              semss[q3]).wait()

    for q in range(4):
        fetch_idx(q, q)
    for q in range(2):
        wait_idx(q, q)
        start_gather(q, q)
    plsc.subcore_barrier()

    def twelve(i, carry):
        for u in range(12):
            j = 12 * i + u

            @pl.when(j < nch)
            def _():
                wait_gather(u % 4, u % 3)
                start_scatter(u % 4, u % 3)

                @pl.when(j + 2 < nch)
                def _():
                    @pl.when(j >= 1)
                    def _():
                        wait_scatter((u + 3) % 4, (u + 2) % 3)

                        @pl.when(j + 3 < nch)
                        def _():
                            fetch_idx(j + 3, (u + 3) % 4)

                    wait_idx(j + 2, (u + 2) % 4)
                    start_gather((u + 2) % 4, (u + 2) % 3)

        return carry

    lax.fori_loop(0, (BASE_CH + 12) // 12, twelve, 0)

    for q in range(3):
        wait_scatter(q, q)

    plsc.subcore_barrier()

    @pl.when(s < NS - 1)
    def _():
        pltpu.sync_copy(acc.at[pl.ds(r0, RPT), :],
                        gp.at[c, pl.ds(r0, RPT), :])

    @pl.when(s == NS - 1)
    def _():
        pltpu.sync_copy(acc.at[pl.ds(r0, RPT_LAST), :],
                        gp.at[c, pl.ds(r0, RPT_LAST), :])


_DEG_SCRATCH = [
    pltpu.VMEM((CH,), jnp.int32),
    pltpu.VMEM((CH,), jnp.int32),
    pltpu.VMEM((CH,), jnp.int32),
    pltpu.VMEM((CH,), jnp.int32),
    pltpu.VMEM((N_PAD,), jnp.float32),
    pltpu.SemaphoreType.DMA,
    pltpu.SemaphoreType.DMA,
    pltpu.SemaphoreType.DMA,
    pltpu.SemaphoreType.DMA,
]
_GS_SCRATCH = [
    pltpu.VMEM((CH,), jnp.int32),
    pltpu.VMEM((CH,), jnp.int32),
    pltpu.VMEM((CH,), jnp.int32),
    pltpu.VMEM((CH,), jnp.int32),
    pltpu.VMEM((CH,), jnp.int32),
    pltpu.VMEM((CH,), jnp.int32),
    pltpu.VMEM((CH,), jnp.int32),
    pltpu.VMEM((CH,), jnp.int32),
    pltpu.VMEM((CH, D), jnp.float32),
    pltpu.VMEM((CH, D), jnp.float32),
    pltpu.VMEM((CH, D), jnp.float32),
    pltpu.VMEM_SHARED((N, D), jnp.float32),
] + [pltpu.SemaphoreType.DMA] * 10

_sc_degree = pl.kernel(
    _sc_degree_body,
    out_type=jax.ShapeDtypeStruct((NW, N_PAD), jnp.float32),
    mesh=_MESH,
    scratch_types=_DEG_SCRATCH,
    compiler_params=pltpu.CompilerParams(needs_layout_passes=False),
)

_sc_gather_scatter = pl.kernel(
    _sc_gather_scatter_body,
    out_type=jax.ShapeDtypeStruct((NC, N, D), jnp.float32),
    mesh=_MESH,
    scratch_types=_GS_SCRATCH,
)


def _dinv_col(degp_blk):
    deg = jnp.sum(degp_blk, axis=0) + 1.0
    return lax.rsqrt(deg)[:, None]


def _tc_mm1_body(x_ref, w1_ref, p1_ref):
    p1_ref[...] = jnp.dot(
        x_ref[...], w1_ref[...], preferred_element_type=jnp.float32
    )


def _tc_scale1_body(degp_ref, p1_ref, hp1_ref):
    hp1_ref[...] = _dinv_col(degp_ref[...]) * p1_ref[...]


def _tc2_body(degp_ref, g_ref, hp1_ref, b1_ref, w2_ref, hp2_ref):
    dinv = _dinv_col(degp_ref[...])
    g = g_ref[0] + g_ref[1]
    h1 = jnp.maximum(dinv * (g + hp1_ref[...]) + b1_ref[...], 0.0)
    hp2_ref[...] = dinv * jnp.dot(
        h1, w2_ref[...], preferred_element_type=jnp.float32
    )


def _tc3_body(degp_ref, g_ref, hp2_ref, b2_ref, out_ref):
    out_ref[...] = (
        _dinv_col(degp_ref[...]) * (g_ref[0] + g_ref[1] + hp2_ref[...])
        + b2_ref[...]
    )


_tc_mm1 = pl.pallas_call(
    _tc_mm1_body,
    grid=(GRIDE,),
    in_specs=[
        pl.BlockSpec((BRE, D), lambda i: (i, 0)),
        pl.BlockSpec((D, D), lambda i: (0, 0)),
    ],
    out_specs=pl.BlockSpec((BRE, D), lambda i: (i, 0)),
    out_shape=jax.ShapeDtypeStruct((N, D), jnp.float32),
)

_tc_scale1 = pl.pallas_call(
    _tc_scale1_body,
    grid=(GRIDE,),
    in_specs=[
        pl.BlockSpec((NW, BRE), lambda i: (0, i)),
        pl.BlockSpec((BRE, D), lambda i: (i, 0)),
    ],
    out_specs=pl.BlockSpec((BRE, D), lambda i: (i, 0)),
    out_shape=jax.ShapeDtypeStruct((N, D), jnp.float32),
)

_tc2 = pl.pallas_call(
    _tc2_body,
    grid=(GRID2,),
    in_specs=[
        pl.BlockSpec((NW, BR2), lambda i: (0, i)),
        pl.BlockSpec((2, BR2, D), lambda i: (0, i, 0)),
        pl.BlockSpec((BR2, D), lambda i: (i, 0)),
        pl.BlockSpec((1, D), lambda i: (0, 0)),
        pl.BlockSpec((D, D), lambda i: (0, 0)),
    ],
    out_specs=pl.BlockSpec((BR2, D), lambda i: (i, 0)),
    out_shape=jax.ShapeDtypeStruct((N, D), jnp.float32),
)

_tc3 = pl.pallas_call(
    _tc3_body,
    grid=(GRID2,),
    in_specs=[
        pl.BlockSpec((NW, BR2), lambda i: (0, i)),
        pl.BlockSpec((2, BR2, D), lambda i: (0, i, 0)),
        pl.BlockSpec((BR2, D), lambda i: (i, 0)),
        pl.BlockSpec((1, D), lambda i: (0, 0)),
    ],
    out_specs=pl.BlockSpec((BR2, D), lambda i: (i, 0)),
    out_shape=jax.ShapeDtypeStruct((N, D), jnp.float32),
)


def kernel(x, edge_index, W1, b1, W2, b2):
    ei = edge_index.astype(jnp.int32)
    zz = jnp.zeros((N, D), jnp.float32)

    degp = _sc_degree(ei)
    p1 = _tc_mm1(x, W1)
    hp1 = _tc_scale1(degp, p1)
    g1p = _sc_gather_scatter(hp1, ei, zz)
    hp2 = _tc2(degp, g1p, hp1, b1.reshape(1, D), W2)
    g2p = _sc_gather_scatter(hp2, ei, zz)
    out = _tc3(degp, g2p, hp2, b2.reshape(1, D))
    return out


# speedup vs baseline: 38.9741x; 1.0240x over previous
import functools

import jax
import jax.numpy as jnp
from jax import lax
from jax.experimental import pallas as pl
from jax.experimental.pallas import tpu as pltpu
from jax.experimental.pallas import tpu_sc as plsc

N = 10000
E = 320000
D = 128

NC = 2
NS = 16
NW = NC * NS
CH = 128
NCHUNKS = E // CH
BASE_CH = NCHUNKS // NW
REM_CH = NCHUNKS % NW

DCH = 2560
NDCH = E // DCH
DBASE = NDCH // NW
DREM = NDCH % NW

N_PAD = 10240
RPT = 640
RPT_LAST = N - 15 * RPT
LANES = 16

BR = 256
GRID = N_PAD // BR
BR2 = 1024
GRID2 = N_PAD // BR2
BRE = 1024
GRIDE = N_PAD // BRE

_MESH = plsc.VectorSubcoreMesh(
    core_axis_name="c", subcore_axis_name="s", num_cores=NC, num_subcores=NS
)


def _sc_degree_body(ei, degp, dst0, dst1, dst2, dst3, degloc,
                    semi0, semi1, semi2, semi3):
    c = lax.axis_index("c")
    s = lax.axis_index("s")
    w = s * NC + c

    dsts = (dst0, dst1, dst2, dst3)
    semis = (semi0, semi1, semi2, semi3)

    def fetch_idx(j, q):
        off = (j * NW + w) * DCH
        pltpu.async_copy(ei.at[1, pl.ds(off, DCH)], dsts[q], semis[q])

    def wait_idx(j, q):
        off = (j * NW + w) * DCH
        pltpu.make_async_copy(ei.at[1, pl.ds(off, DCH)], dsts[q],
                              semis[q]).wait()

    nch = DBASE + jnp.where(w < DREM, 1, 0)

    for q in range(4):
        @pl.when(q < nch)
        def _():
            fetch_idx(q, q)

    zeros = jnp.zeros((LANES,), jnp.float32)

    def zinit(i, carry):
        degloc[pl.ds(i * LANES, LANES)] = zeros
        return carry

    lax.fori_loop(0, N_PAD // LANES, zinit, 0)

    ones = jnp.ones((LANES,), jnp.float32)

    for q in range(4):
        @pl.when(q < nch)
        def _():
            wait_idx(q, q)

            def group(k, carry2):
                idx = dsts[q][pl.ds(k * LANES, LANES)]
                plsc.addupdate_scatter(degloc, [idx], ones)
                return carry2

            lax.fori_loop(0, DCH // LANES, group, 0)

    pltpu.sync_copy(degloc, degp.at[w])


def _sc_gather_scatter_body(hp, ei, zz, gp,
                            src0, src1, src2, src3,
                            dst0, dst1, dst2, dst3,
                            rows0, rows1, rows2, acc,
                            semg0, semg1, semg2,
                            sems0, sems1, sems2,
                            semi0, semi1, semi2, semi3):
    c = lax.axis_index("c")
    s = lax.axis_index("s")
    w = s * NC + c
    r0 = s * RPT
    nch = BASE_CH + jnp.where(w < REM_CH, 1, 0)

    @pl.when(s < NS - 1)
    def _():
        pltpu.sync_copy(zz.at[pl.ds(r0, RPT), :], acc.at[pl.ds(r0, RPT), :])

    @pl.when(s == NS - 1)
    def _():
        pltpu.sync_copy(zz.at[pl.ds(r0, RPT_LAST), :],
                        acc.at[pl.ds(r0, RPT_LAST), :])

    rowss = (rows0, rows1, rows2)
    semgs = (semg0, semg1, semg2)
    semss = (sems0, sems1, sems2)
    semis = (semi0, semi1, semi2, semi3)
    srcs = (src0, src1, src2, src3)
    dsts = (dst0, dst1, dst2, dst3)

    def fetch_idx(j, q4):
        off = (j * NW + w) * CH
        pltpu.async_copy(ei.at[0, pl.ds(off, CH)], srcs[q4], semis[q4])
        pltpu.async_copy(ei.at[1, pl.ds(off, CH)], dsts[q4], semis[q4])

    def wait_idx(j, q4):
        off = (j * NW + w) * CH
        pltpu.make_async_copy(ei.at[0, pl.ds(off, CH)], srcs[q4],
                              semis[q4]).wait()
        pltpu.make_async_copy(ei.at[1, pl.ds(off, CH)], dsts[q4],
                              semis[q4]).wait()

    def start_gather(q4, q3):
        pltpu.async_copy(hp.at[srcs[q4]], rowss[q3], semgs[q3])

    def wait_gather(q4, q3):
        pltpu.make_async_copy(hp.at[srcs[q4]], rowss[q3], semgs[q3]).wait()

    def start_scatter(q4, q3):
        pltpu.async_copy(rowss[q3], acc.at[dsts[q4]], semss[q3], add=True)

    def wait_scatter(q4, q3):
        pltpu.make_async_copy(rowss[q3], acc.at[dsts[q4]],
                              semss[q3]).wait()

    for q in range(4):
        fetch_idx(q, q)
    for q in range(2):
        wait_idx(q, q)
        start_gather(q, q)
    plsc.subcore_barrier()

    def twelve(i, carry):
        for u in range(12):
            j = 12 * i + u

            @pl.when(j < nch)
            def _():
                wait_gather(u % 4, u % 3)
                start_scatter(u % 4, u % 3)

                @pl.when(j + 2 < nch)
                def _():
                    @pl.when(j >= 1)
                    def _():
                        wait_scatter((u + 3) % 4, (u + 2) % 3)

                        @pl.when(j + 3 < nch)
                        def _():
                            fetch_idx(j + 3, (u + 3) % 4)

                    wait_idx(j + 2, (u + 2) % 4)
                    start_gather((u + 2) % 4, (u + 2) % 3)

        return carry

    lax.fori_loop(0, (BASE_CH + 12) // 12, twelve, 0)

    for q in range(3):
        wait_scatter(q, q)

    plsc.subcore_barrier()

    @pl.when(s < NS - 1)
    def _():
        pltpu.sync_copy(acc.at[pl.ds(r0, RPT), :],
                        gp.at[c, pl.ds(r0, RPT), :])

    @pl.when(s == NS - 1)
    def _():
        pltpu.sync_copy(acc.at[pl.ds(r0, RPT_LAST), :],
                        gp.at[c, pl.ds(r0, RPT_LAST), :])


_DEG_SCRATCH = [
    pltpu.VMEM((DCH,), jnp.int32),
    pltpu.VMEM((DCH,), jnp.int32),
    pltpu.VMEM((DCH,), jnp.int32),
    pltpu.VMEM((DCH,), jnp.int32),
    pltpu.VMEM((N_PAD,), jnp.float32),
    pltpu.SemaphoreType.DMA,
    pltpu.SemaphoreType.DMA,
    pltpu.SemaphoreType.DMA,
    pltpu.SemaphoreType.DMA,
]
_GS_SCRATCH = [
    pltpu.VMEM((CH,), jnp.int32),
    pltpu.VMEM((CH,), jnp.int32),
    pltpu.VMEM((CH,), jnp.int32),
    pltpu.VMEM((CH,), jnp.int32),
    pltpu.VMEM((CH,), jnp.int32),
    pltpu.VMEM((CH,), jnp.int32),
    pltpu.VMEM((CH,), jnp.int32),
    pltpu.VMEM((CH,), jnp.int32),
    pltpu.VMEM((CH, D), jnp.float32),
    pltpu.VMEM((CH, D), jnp.float32),
    pltpu.VMEM((CH, D), jnp.float32),
    pltpu.VMEM_SHARED((N, D), jnp.float32),
] + [pltpu.SemaphoreType.DMA] * 10

_sc_degree = pl.kernel(
    _sc_degree_body,
    out_type=jax.ShapeDtypeStruct((NW, N_PAD), jnp.float32),
    mesh=_MESH,
    scratch_types=_DEG_SCRATCH,
    compiler_params=pltpu.CompilerParams(needs_layout_passes=False),
)

_sc_gather_scatter = pl.kernel(
    _sc_gather_scatter_body,
    out_type=jax.ShapeDtypeStruct((NC, N, D), jnp.float32),
    mesh=_MESH,
    scratch_types=_GS_SCRATCH,
)


def _dinv_col(degp_blk):
    deg = jnp.sum(degp_blk, axis=0) + 1.0
    return lax.rsqrt(deg)[:, None]


def _tc_mm1_body(x_ref, w1_ref, p1_ref):
    p1_ref[...] = jnp.dot(
        x_ref[...], w1_ref[...], preferred_element_type=jnp.float32
    )


def _tc_scale1_body(degp_ref, p1_ref, hp1_ref):
    hp1_ref[...] = _dinv_col(degp_ref[...]) * p1_ref[...]


def _tc2_body(degp_ref, g_ref, hp1_ref, b1_ref, w2_ref, hp2_ref):
    dinv = _dinv_col(degp_ref[...])
    g = g_ref[0] + g_ref[1]
    h1 = jnp.maximum(dinv * (g + hp1_ref[...]) + b1_ref[...], 0.0)
    hp2_ref[...] = dinv * jnp.dot(
        h1, w2_ref[...], preferred_element_type=jnp.float32
    )


def _tc3_body(degp_ref, g_ref, hp2_ref, b2_ref, out_ref):
    out_ref[...] = (
        _dinv_col(degp_ref[...]) * (g_ref[0] + g_ref[1] + hp2_ref[...])
        + b2_ref[...]
    )


_tc_mm1 = pl.pallas_call(
    _tc_mm1_body,
    grid=(GRIDE,),
    in_specs=[
        pl.BlockSpec((BRE, D), lambda i: (i, 0)),
        pl.BlockSpec((D, D), lambda i: (0, 0)),
    ],
    out_specs=pl.BlockSpec((BRE, D), lambda i: (i, 0)),
    out_shape=jax.ShapeDtypeStruct((N, D), jnp.float32),
)

_tc_scale1 = pl.pallas_call(
    _tc_scale1_body,
    grid=(GRIDE,),
    in_specs=[
        pl.BlockSpec((NW, BRE), lambda i: (0, i)),
        pl.BlockSpec((BRE, D), lambda i: (i, 0)),
    ],
    out_specs=pl.BlockSpec((BRE, D), lambda i: (i, 0)),
    out_shape=jax.ShapeDtypeStruct((N, D), jnp.float32),
)

_tc2 = pl.pallas_call(
    _tc2_body,
    grid=(GRID2,),
    in_specs=[
        pl.BlockSpec((NW, BR2), lambda i: (0, i)),
        pl.BlockSpec((2, BR2, D), lambda i: (0, i, 0)),
        pl.BlockSpec((BR2, D), lambda i: (i, 0)),
        pl.BlockSpec((1, D), lambda i: (0, 0)),
        pl.BlockSpec((D, D), lambda i: (0, 0)),
    ],
    out_specs=pl.BlockSpec((BR2, D), lambda i: (i, 0)),
    out_shape=jax.ShapeDtypeStruct((N, D), jnp.float32),
)

_tc3 = pl.pallas_call(
    _tc3_body,
    grid=(GRID2,),
    in_specs=[
        pl.BlockSpec((NW, BR2), lambda i: (0, i)),
        pl.BlockSpec((2, BR2, D), lambda i: (0, i, 0)),
        pl.BlockSpec((BR2, D), lambda i: (i, 0)),
        pl.BlockSpec((1, D), lambda i: (0, 0)),
    ],
    out_specs=pl.BlockSpec((BR2, D), lambda i: (i, 0)),
    out_shape=jax.ShapeDtypeStruct((N, D), jnp.float32),
)


def kernel(x, edge_index, W1, b1, W2, b2):
    zz = jnp.zeros((N, D), jnp.float32)
    ei = edge_index.astype(jnp.int32)

    degp = _sc_degree(ei)
    p1 = _tc_mm1(x, W1)
    hp1 = _tc_scale1(degp, p1)
    g1p = _sc_gather_scatter(hp1, ei, zz)
    hp2 = _tc2(degp, g1p, hp1, b1.reshape(1, D), W2)
    g2p = _sc_gather_scatter(hp2, ei, zz)
    out = _tc3(degp, g2p, hp2, b2.reshape(1, D))
    return out
